# baseline (device time: 140877 ns/iter reference)
import jax
import jax.numpy as jnp
from jax import lax
from jax.experimental import pallas as pl
from jax.experimental.pallas import tpu as pltpu

N_Y = 4
N_SLAB = 2
F_SLAB = 1024


def kernel(x, dy):
    m_shard, d = x.shape
    _, f = dy.shape
    chunk = d // N_Y
    part_w = f // 4


    def body(xb_ref, dy_ref, out_ref, dyf32, stage, yrecv, xsend, xrecv,
             ostage, dy_sems, ysend_sems, yrecv_sems, xsend_sems, xrecv_sems,
             out_sems):
        y = lax.axis_index("y")
        xp = lax.axis_index("x")
        zp = lax.axis_index("z")
        left = (y + N_Y - 1) % N_Y
        right = (y + 1) % N_Y
        zq = zp % 2
        zn = zp + 1 - 2 * zq
        part = 2 * zq + xp
        base = part * part_w

        partners = [
            ((1 - xp, y, zp), 2 * zq + (1 - xp)),
            ((xp, y, zn), 2 * (1 - zq) + xp),
            ((1 - xp, y, zn), 2 * (1 - zq) + (1 - xp)),
        ]

        barrier = pltpu.get_barrier_semaphore()
        for dev in [(xp, left, zp), (xp, right, zp)] + [p[0] for p in partners]:
            pl.semaphore_signal(barrier, inc=1, device_id=dev,
                                device_id_type=pl.DeviceIdType.MESH)
        pl.semaphore_wait(barrier, 5)

        def dy_load(s):
            return pltpu.make_async_copy(
                dy_ref.at[:, pl.ds(base + s * F_SLAB, F_SLAB)],
                dyf32.at[s], dy_sems.at[s])

        dy_load(0).start()
        dy_load(1).start()

        WAVE = [(s, h) for h in range(N_Y - 1) for s in range(N_SLAB)]

        def make_yrdma(g):
            s, h = WAVE[g]
            return pltpu.make_async_remote_copy(
                src_ref=stage.at[g % 2],
                dst_ref=yrecv.at[s, h],
                send_sem=ysend_sems.at[g % 2],
                recv_sem=yrecv_sems.at[s, h],
                device_id=(xp, right, zp),
                device_id_type=pl.DeviceIdType.MESH,
            )

        def make_yrecv_wait(s, h):
            return pltpu.make_async_remote_copy(
                src_ref=stage.at[0],
                dst_ref=yrecv.at[s, h],
                send_sem=ysend_sems.at[0],
                recv_sem=yrecv_sems.at[s, h],
                device_id=(xp, right, zp),
                device_id_type=pl.DeviceIdType.MESH,
            )

        def make_xrdma(s, j):
            return pltpu.make_async_remote_copy(
                src_ref=xsend,
                dst_ref=xrecv.at[s, j],
                send_sem=xsend_sems.at[s, j],
                recv_sem=xrecv_sems.at[s, j],
                device_id=partners[j][0],
                device_id_type=pl.DeviceIdType.MESH,
            )

        def dot_chunk(c, s):
            xcols = xb_ref[:, pl.ds(c * chunk, chunk)]
            return lax.dot_general(
                xcols, dyf32[s, :, :],
                (((0,), (0,)), ((), ())),
                preferred_element_type=jnp.float32,
            )

        dy_ready = set()
        for g, (s, h) in enumerate(WAVE):
            if s not in dy_ready:
                dy_load(s).wait()
                dy_ready.add(s)
            c = (y + (N_Y - 1 - h)) % N_Y
            p = dot_chunk(c, s)
            if h > 0:
                make_yrecv_wait(s, h - 1).wait_recv()
                p = p + yrecv[s, h - 1, :, :].astype(jnp.float32)
            if g >= 2:
                make_yrdma(g - 2).wait_send()
            stage[g % 2, :, :] = p.astype(jnp.bfloat16)
            make_yrdma(g).start()

        for s in range(N_SLAB):
            p_own = dot_chunk(y, s)
            make_yrecv_wait(s, N_Y - 2).wait_recv()
            fin = p_own + yrecv[s, N_Y - 2, :, :].astype(jnp.float32)

            ostage[s, :, :] = fin
            pltpu.make_async_copy(
                ostage.at[s],
                out_ref.at[:, pl.ds(base + s * F_SLAB, F_SLAB)],
                out_sems.at[s],
            ).start()

            if s > 0:
                for j in range(3):
                    make_xrdma(s - 1, j).wait_send()
            xsend[:, :] = fin.astype(jnp.bfloat16)
            for j in range(3):
                make_xrdma(s, j).start()

        for s in range(N_SLAB):
            for j in range(3):
                make_xrdma(s, j).wait_recv()
                q = (s * 3 + j) % 2
                pltpu.make_async_copy(
                    ostage.at[q], out_ref.at[:, pl.ds(0, F_SLAB)],
                    out_sems.at[q],
                ).wait()
                ostage[q, :, :] = xrecv[s, j, :, :].astype(jnp.float32)
                col = partners[j][1] * part_w + s * F_SLAB
                pltpu.make_async_copy(
                    ostage.at[q],
                    out_ref.at[:, pl.ds(col, F_SLAB)],
                    out_sems.at[q],
                ).start()

        total_g = len(WAVE)
        for g in (total_g - 2, total_g - 1):
            make_yrdma(g).wait_send()
        for j in range(3):
            make_xrdma(N_SLAB - 1, j).wait_send()
        for q in range(2):
            pltpu.make_async_copy(
                ostage.at[q], out_ref.at[:, pl.ds(0, F_SLAB)], out_sems.at[q]
            ).wait()

    return pl.pallas_call(
        body,
        out_shape=jax.ShapeDtypeStruct((chunk, f), jnp.float32),
        in_specs=[
            pl.BlockSpec(memory_space=pltpu.VMEM),
            pl.BlockSpec(memory_space=pl.ANY),
        ],
        out_specs=pl.BlockSpec(memory_space=pl.ANY),
        scratch_shapes=[
            pltpu.VMEM((N_SLAB, m_shard, F_SLAB), jnp.float32),
            pltpu.VMEM((2, chunk, F_SLAB), jnp.bfloat16),
            pltpu.VMEM((N_SLAB, N_Y - 1, chunk, F_SLAB), jnp.bfloat16),
            pltpu.VMEM((chunk, F_SLAB), jnp.bfloat16),
            pltpu.VMEM((N_SLAB, 3, chunk, F_SLAB), jnp.bfloat16),
            pltpu.VMEM((2, chunk, F_SLAB), jnp.float32),
            pltpu.SemaphoreType.DMA((N_SLAB,)),
            pltpu.SemaphoreType.DMA((2,)),
            pltpu.SemaphoreType.DMA((N_SLAB, N_Y - 1)),
            pltpu.SemaphoreType.DMA((N_SLAB, 3)),
            pltpu.SemaphoreType.DMA((N_SLAB, 3)),
            pltpu.SemaphoreType.DMA((2,)),
        ],
        compiler_params=pltpu.CompilerParams(
            collective_id=0,
            vmem_limit_bytes=66_846_720,
        ),
    )(x, dy)
